# baseline (speedup 1.0000x reference)
import functools

import jax
import jax.numpy as jnp
from jax import lax
from jax.experimental import pallas as pl
from jax.experimental.pallas import tpu as pltpu
from jax.experimental.pallas import tpu_sc as plsc

N1 = 8192
N2 = 8192
D = 256
M = 8192

BM = 2048
BN = 2048
GI = N1 // BM
GJ = N2 // BN

_INF = float("inf")


def _merge2(p1, p2, q1, q2):
    n1 = jnp.minimum(p1, q1)
    n2 = jnp.minimum(jnp.maximum(p1, q1), jnp.minimum(p2, q2))
    return n1, n2


def _snn_body(d1_ref, d2_ref, snn1_ref, snn2_ref, rm1, rm2, cm1, cm2):
    i = pl.program_id(0)
    j = pl.program_id(1)
    d1 = d1_ref[...]
    d2 = d2_ref[...]
    p = lax.dot_general(d1, d2, (((1,), (1,)), ((), ())),
                        preferred_element_type=jnp.float32)
    a2h = 0.5 * jnp.sum(d1 * d1, axis=1, keepdims=True)
    b2h = 0.5 * jnp.sum(d2 * d2, axis=1)[None, :]

    w = (a2h - p) + b2h

    m1 = jnp.min(w, axis=1, keepdims=True)
    m2 = jnp.min(jnp.where(w == m1, _INF, w), axis=1, keepdims=True)
    o1 = jnp.where(j == 0, _INF, rm1[...])
    o2 = jnp.where(j == 0, _INF, rm2[...])
    n1, n2 = _merge2(o1, o2, m1, m2)
    rm1[...] = n1
    rm2[...] = n2

    @pl.when(j == GJ - 1)
    def _():
        snn1_ref[...] = jnp.sqrt(jnp.maximum(2.0 * n2, 0.0))

    k1 = jnp.min(w, axis=0, keepdims=True)
    k2 = jnp.min(jnp.where(w == k1, _INF, w), axis=0, keepdims=True)
    q1 = jnp.where(i == 0, _INF, cm1[0:1, pl.ds(j * BN, BN)])
    q2 = jnp.where(i == 0, _INF, cm2[0:1, pl.ds(j * BN, BN)])
    z1, z2 = _merge2(q1, q2, k1, k2)
    cm1[0:1, pl.ds(j * BN, BN)] = z1
    cm2[0:1, pl.ds(j * BN, BN)] = z2

    @pl.when(i == GI - 1)
    def _():
        snn2_ref[...] = jnp.sqrt(jnp.maximum(2.0 * z2, 0.0))


def _snn_call(desc1, desc2, interpret=False):
    return pl.pallas_call(
        _snn_body,
        grid=(GI, GJ),
        in_specs=[
            pl.BlockSpec((BM, D), lambda i, j: (i, 0)),
            pl.BlockSpec((BN, D), lambda i, j: (j, 0)),
        ],
        out_specs=[
            pl.BlockSpec((BM, 1), lambda i, j: (i, 0)),
            pl.BlockSpec((1, BN), lambda i, j: (0, j)),
        ],
        out_shape=[
            jax.ShapeDtypeStruct((N1, 1), jnp.float32),
            jax.ShapeDtypeStruct((1, N2), jnp.float32),
        ],
        scratch_shapes=[
            pltpu.VMEM((BM, 1), jnp.float32),
            pltpu.VMEM((BM, 1), jnp.float32),
            pltpu.VMEM((1, N2), jnp.float32),
            pltpu.VMEM((1, N2), jnp.float32),
        ],
        compiler_params=pltpu.CompilerParams(
            dimension_semantics=("arbitrary", "arbitrary")),
        interpret=interpret,
    )(desc1, desc2)


_NW = 32
_BPW = M // _NW
_HALF = _BPW // 2


def _gather_body(mflat_hbm, d1_hbm, d2_hbm, t1_hbm, t2_hbm, l1_hbm, l2_hbm,
                 arows_hbm, brows_hbm, hard_hbm, lgm_hbm,
                 pairs_v, idx0_v, idx1_v, t1_v, t2_v, l1_v, l2_v,
                 rows1_v, rows2_v, hard_v, lgm_v, sem0, sem1):
    wid = lax.axis_index("s") * 2 + lax.axis_index("c")
    pltpu.sync_copy(mflat_hbm.at[pl.ds(wid * 4, 4)], pairs_v)
    pltpu.sync_copy(t1_hbm, t1_v)
    pltpu.sync_copy(t2_hbm, t2_v)
    pltpu.sync_copy(l1_hbm, l1_v)
    pltpu.sync_copy(l2_hbm, l2_v)

    lanes = jnp.arange(16, dtype=jnp.int32)
    for g in range(_BPW // 16):
        pos = lanes * 2 + (g * 32)
        phi = lax.shift_right_logical(pos, 7)
        plo = pos & 127
        i0 = plsc.load_gather(pairs_v, [phi, plo])
        i1 = plsc.load_gather(pairs_v, [phi, plo + 1])
        h = g // (_HALF // 16)
        off = (g % (_HALF // 16)) * 16
        idx0_v[h, pl.ds(off, 16)] = i0
        idx1_v[h, pl.ds(off, 16)] = i1
        i0h = lax.shift_right_logical(i0, 7)
        i0l = i0 & 127
        i1h = lax.shift_right_logical(i1, 7)
        i1l = i1 & 127
        s1 = plsc.load_gather(t1_v, [i0h, i0l])
        s2 = plsc.load_gather(t2_v, [i1h, i1l])
        hard_v[g, :] = jnp.minimum(s1, s2)
        a1 = plsc.load_gather(l1_v, [i0h, i0l])
        a2 = plsc.load_gather(l2_v, [i1h, i1l])
        lgm_v[g, :] = jnp.minimum(a1, a2)

    pltpu.sync_copy(hard_v, hard_hbm.at[wid])
    pltpu.sync_copy(lgm_v, lgm_hbm.at[wid])

    base = wid * _BPW
    for h in range(2):
        cp0 = pltpu.async_copy(d1_hbm.at[idx0_v.at[h]], rows1_v, sem0)
        cp1 = pltpu.async_copy(d2_hbm.at[idx1_v.at[h]], rows2_v, sem1)
        cp0.wait()
        cp1.wait()
        pltpu.sync_copy(rows1_v, arows_hbm.at[pl.ds(base + h * _HALF, _HALF)])
        pltpu.sync_copy(rows2_v, brows_hbm.at[pl.ds(base + h * _HALF, _HALF)])


def _gather_call(matches_flat, desc1, desc2, snn1, snn2, logits_1, logits_2):
    mesh = plsc.VectorSubcoreMesh(core_axis_name="c", subcore_axis_name="s")
    fn = pl.kernel(
        _gather_body,
        out_type=[
            jax.ShapeDtypeStruct((M, D), jnp.float32),
            jax.ShapeDtypeStruct((M, D), jnp.float32),
            jax.ShapeDtypeStruct((_NW, _BPW // 16, 16), jnp.float32),
            jax.ShapeDtypeStruct((_NW, _BPW // 16, 16), jnp.float32),
        ],
        mesh=mesh,
        compiler_params=pltpu.CompilerParams(needs_layout_passes=False),
        scratch_types=[
            pltpu.VMEM((4, 128), jnp.int32),
            pltpu.VMEM((2, _HALF), jnp.int32),
            pltpu.VMEM((2, _HALF), jnp.int32),
            pltpu.VMEM((N1 // 128, 128), jnp.float32),
            pltpu.VMEM((N2 // 128, 128), jnp.float32),
            pltpu.VMEM((N1 // 128, 128), jnp.float32),
            pltpu.VMEM((N2 // 128, 128), jnp.float32),
            pltpu.VMEM((_HALF, D), jnp.float32),
            pltpu.VMEM((_HALF, D), jnp.float32),
            pltpu.VMEM((_BPW // 16, 16), jnp.float32),
            pltpu.VMEM((_BPW // 16, 16), jnp.float32),
            pltpu.SemaphoreType.DMA,
            pltpu.SemaphoreType.DMA,
        ],
    )
    return fn(matches_flat.reshape(_NW * 4, 128), desc1, desc2,
              snn1.reshape(N1 // 128, 128), snn2.reshape(N2 // 128, 128),
              logits_1.reshape(N1 // 128, 128), logits_2.reshape(N2 // 128, 128))


_RB = 1024
_GC = M // _RB


def _loss_body(a_ref, b_ref, hard_ref, lg_ref, inl_ref, out_ref, acc):
    k = pl.program_id(0)
    a = a_ref[...]
    b = b_ref[...]
    d = a - b + 1e-6
    dp = jnp.sqrt(jnp.sum(d * d, axis=1))
    hard = hard_ref[0, :]
    lg = lg_ref[0, :]
    w = inl_ref[0, :]
    pos = jnp.maximum(1.0 + dp - hard, 0.0)
    neg = jnp.maximum(1.0 - dp, 0.0)
    sums = [
        jnp.sum(pos * lg * w), jnp.sum(pos * lg),
        jnp.sum(neg * lg * w), jnp.sum(neg * lg),
        jnp.sum(lg * w), jnp.sum(lg), jnp.sum(w),
    ]
    for n, s in enumerate(sums):
        prev = jnp.where(k == 0, 0.0, acc[n])
        acc[n] = prev + s
    @pl.when(k == _GC - 1)
    def _():
        for n in range(7):
            out_ref[n] = acc[n]


def _loss_call(a_rows, b_rows, hard, lg, inl, interpret=False):
    return pl.pallas_call(
        _loss_body,
        grid=(_GC,),
        in_specs=[
            pl.BlockSpec((_RB, D), lambda k: (k, 0)),
            pl.BlockSpec((_RB, D), lambda k: (k, 0)),
            pl.BlockSpec((1, _RB), lambda k: (0, k)),
            pl.BlockSpec((1, _RB), lambda k: (0, k)),
            pl.BlockSpec((1, _RB), lambda k: (0, k)),
        ],
        out_specs=pl.BlockSpec(memory_space=pltpu.SMEM),
        out_shape=jax.ShapeDtypeStruct((7,), jnp.float32),
        scratch_shapes=[pltpu.SMEM((7,), jnp.float32)],
        compiler_params=pltpu.CompilerParams(
            dimension_semantics=("arbitrary",)),
        interpret=interpret,
    )(a_rows, b_rows, hard, lg, inl)


def _assemble(sums, label):
    use_ones = sums[6] < 8.0
    pos_s = jnp.where(use_ones, sums[1], sums[0])
    neg_s = jnp.where(use_ones, sums[3], sums[2])
    lg_s = jnp.where(use_ones, sums[5], sums[4])
    num = jnp.where(label != 0, pos_s, neg_s)
    return num / (lg_s + 1e-8)


def kernel(desc1, desc2, matches, inliers, label, logits_1, logits_2):
    snn1, snn2 = _snn_call(desc1, desc2)
    a_rows, b_rows, hard, lgm = _gather_call(
        matches.reshape(-1), desc1, desc2,
        snn1.reshape(-1), snn2.reshape(-1), logits_1, logits_2)
    sums = _loss_call(
        a_rows, b_rows, hard.reshape(1, M), lgm.reshape(1, M),
        inliers.astype(jnp.float32).reshape(1, M))
    return _assemble(sums, label)


# speedup vs baseline: 1.0587x; 1.0587x over previous
import functools

import jax
import jax.numpy as jnp
from jax import lax
from jax.experimental import pallas as pl
from jax.experimental.pallas import tpu as pltpu
from jax.experimental.pallas import tpu_sc as plsc

N1 = 8192
N2 = 8192
D = 256
M = 8192

BM = 2048
BN = 4096
GI = N1 // BM
GJ = N2 // BN

_INF = float("inf")


def _merge2(p1, p2, q1, q2):
    n1 = jnp.minimum(p1, q1)
    n2 = jnp.minimum(jnp.maximum(p1, q1), jnp.minimum(p2, q2))
    return n1, n2


def _snn_body(d1_ref, d2_ref, snn1_ref, snn2_ref, rm1, rm2, cm1, cm2):
    i = pl.program_id(0)
    j = pl.program_id(1)
    d1 = d1_ref[...]
    d2 = d2_ref[...]
    p = lax.dot_general(d1, d2, (((1,), (1,)), ((), ())),
                        preferred_element_type=jnp.float32)
    a2h = 0.5 * jnp.sum(d1 * d1, axis=1, keepdims=True)
    b2h = 0.5 * jnp.sum(d2 * d2, axis=1)[None, :]

    w = (a2h - p) + b2h

    m1 = jnp.min(w, axis=1, keepdims=True)
    m2 = jnp.min(jnp.where(w == m1, _INF, w), axis=1, keepdims=True)
    o1 = jnp.where(j == 0, _INF, rm1[...])
    o2 = jnp.where(j == 0, _INF, rm2[...])
    n1, n2 = _merge2(o1, o2, m1, m2)
    rm1[...] = n1
    rm2[...] = n2

    @pl.when(j == GJ - 1)
    def _():
        snn1_ref[...] = jnp.sqrt(jnp.maximum(2.0 * n2, 0.0))

    k1 = jnp.min(w, axis=0, keepdims=True)
    k2 = jnp.min(jnp.where(w == k1, _INF, w), axis=0, keepdims=True)
    q1 = jnp.where(i == 0, _INF, cm1[0:1, pl.ds(j * BN, BN)])
    q2 = jnp.where(i == 0, _INF, cm2[0:1, pl.ds(j * BN, BN)])
    z1, z2 = _merge2(q1, q2, k1, k2)
    cm1[0:1, pl.ds(j * BN, BN)] = z1
    cm2[0:1, pl.ds(j * BN, BN)] = z2

    @pl.when(i == GI - 1)
    def _():
        snn2_ref[...] = jnp.sqrt(jnp.maximum(2.0 * z2, 0.0))


def _snn_call(desc1, desc2, interpret=False):
    return pl.pallas_call(
        _snn_body,
        grid=(GI, GJ),
        in_specs=[
            pl.BlockSpec((BM, D), lambda i, j: (i, 0)),
            pl.BlockSpec((BN, D), lambda i, j: (j, 0)),
        ],
        out_specs=[
            pl.BlockSpec((BM, 1), lambda i, j: (i, 0)),
            pl.BlockSpec((1, BN), lambda i, j: (0, j)),
        ],
        out_shape=[
            jax.ShapeDtypeStruct((N1, 1), jnp.float32),
            jax.ShapeDtypeStruct((1, N2), jnp.float32),
        ],
        scratch_shapes=[
            pltpu.VMEM((BM, 1), jnp.float32),
            pltpu.VMEM((BM, 1), jnp.float32),
            pltpu.VMEM((1, N2), jnp.float32),
            pltpu.VMEM((1, N2), jnp.float32),
        ],
        compiler_params=pltpu.CompilerParams(
            dimension_semantics=("arbitrary", "arbitrary")),
        interpret=interpret,
    )(desc1, desc2)


_NW = 32
_BPW = M // _NW
_HALF = _BPW // 2


def _deinterleave(pairs_v, g, lanes):
    pos = lanes * 2 + (g * 32)
    phi = lax.shift_right_logical(pos, 7)
    plo = pos & 127
    i0 = plsc.load_gather(pairs_v, [phi, plo])
    i1 = plsc.load_gather(pairs_v, [phi, plo + 1])
    return i0, i1


def _rowgather_body(mflat_hbm, d1_hbm, d2_hbm, arows_hbm, brows_hbm,
                    pairs_v, idx0_v, idx1_v, rows1_v, rows2_v, sem0, sem1):
    wid = lax.axis_index("s") * 2 + lax.axis_index("c")
    pltpu.sync_copy(mflat_hbm.at[pl.ds(wid * 4, 4)], pairs_v)
    lanes = jnp.arange(16, dtype=jnp.int32)
    for g in range(_BPW // 16):
        i0, i1 = _deinterleave(pairs_v, g, lanes)
        h = g // (_HALF // 16)
        off = (g % (_HALF // 16)) * 16
        idx0_v[h, pl.ds(off, 16)] = i0
        idx1_v[h, pl.ds(off, 16)] = i1

    base = wid * _BPW
    for h in range(2):
        cp0 = pltpu.async_copy(d1_hbm.at[idx0_v.at[h]], rows1_v, sem0)
        cp1 = pltpu.async_copy(d2_hbm.at[idx1_v.at[h]], rows2_v, sem1)
        cp0.wait()
        cp1.wait()
        pltpu.sync_copy(rows1_v, arows_hbm.at[pl.ds(base + h * _HALF, _HALF)])
        pltpu.sync_copy(rows2_v, brows_hbm.at[pl.ds(base + h * _HALF, _HALF)])


def _rowgather_call(matches_flat, desc1, desc2):
    mesh = plsc.VectorSubcoreMesh(core_axis_name="c", subcore_axis_name="s")
    fn = pl.kernel(
        _rowgather_body,
        out_type=[
            jax.ShapeDtypeStruct((M, D), jnp.float32),
            jax.ShapeDtypeStruct((M, D), jnp.float32),
        ],
        mesh=mesh,
        compiler_params=pltpu.CompilerParams(needs_layout_passes=False),
        scratch_types=[
            pltpu.VMEM((4, 128), jnp.int32),
            pltpu.VMEM((2, _HALF), jnp.int32),
            pltpu.VMEM((2, _HALF), jnp.int32),
            pltpu.VMEM((_HALF, D), jnp.float32),
            pltpu.VMEM((_HALF, D), jnp.float32),
            pltpu.SemaphoreType.DMA,
            pltpu.SemaphoreType.DMA,
        ],
    )
    return fn(matches_flat.reshape(_NW * 4, 128), desc1, desc2)


def _elemgather_body(mflat_hbm, t1_hbm, t2_hbm, l1_hbm, l2_hbm,
                     hard_hbm, lgm_hbm,
                     pairs_v, t1_v, t2_v, l1_v, l2_v, hard_v, lgm_v):
    wid = lax.axis_index("s") * 2 + lax.axis_index("c")
    pltpu.sync_copy(mflat_hbm.at[pl.ds(wid * 4, 4)], pairs_v)
    pltpu.sync_copy(t1_hbm, t1_v)
    pltpu.sync_copy(t2_hbm, t2_v)
    pltpu.sync_copy(l1_hbm, l1_v)
    pltpu.sync_copy(l2_hbm, l2_v)

    lanes = jnp.arange(16, dtype=jnp.int32)
    for g in range(_BPW // 16):
        i0, i1 = _deinterleave(pairs_v, g, lanes)
        i0h = lax.shift_right_logical(i0, 7)
        i0l = i0 & 127
        i1h = lax.shift_right_logical(i1, 7)
        i1l = i1 & 127
        s1 = plsc.load_gather(t1_v, [i0h, i0l])
        s2 = plsc.load_gather(t2_v, [i1h, i1l])
        hard_v[g, :] = jnp.minimum(s1, s2)
        a1 = plsc.load_gather(l1_v, [i0h, i0l])
        a2 = plsc.load_gather(l2_v, [i1h, i1l])
        lgm_v[g, :] = jnp.minimum(a1, a2)

    pltpu.sync_copy(hard_v, hard_hbm.at[wid])
    pltpu.sync_copy(lgm_v, lgm_hbm.at[wid])


def _elemgather_call(matches_flat, snn1, snn2, logits_1, logits_2):
    mesh = plsc.VectorSubcoreMesh(core_axis_name="c", subcore_axis_name="s")
    fn = pl.kernel(
        _elemgather_body,
        out_type=[
            jax.ShapeDtypeStruct((_NW, _BPW // 16, 16), jnp.float32),
            jax.ShapeDtypeStruct((_NW, _BPW // 16, 16), jnp.float32),
        ],
        mesh=mesh,
        compiler_params=pltpu.CompilerParams(needs_layout_passes=False),
        scratch_types=[
            pltpu.VMEM((4, 128), jnp.int32),
            pltpu.VMEM((N1 // 128, 128), jnp.float32),
            pltpu.VMEM((N2 // 128, 128), jnp.float32),
            pltpu.VMEM((N1 // 128, 128), jnp.float32),
            pltpu.VMEM((N2 // 128, 128), jnp.float32),
            pltpu.VMEM((_BPW // 16, 16), jnp.float32),
            pltpu.VMEM((_BPW // 16, 16), jnp.float32),
        ],
    )
    return fn(matches_flat.reshape(_NW * 4, 128),
              snn1.reshape(N1 // 128, 128), snn2.reshape(N2 // 128, 128),
              logits_1.reshape(N1 // 128, 128), logits_2.reshape(N2 // 128, 128))


_RB = 1024
_GC = M // _RB


def _loss_body(a_ref, b_ref, hard_ref, lg_ref, inl_ref, lab_ref, out_ref, acc):
    k = pl.program_id(0)
    a = a_ref[...]
    b = b_ref[...]
    d = a - b + 1e-6
    dp = jnp.sqrt(jnp.sum(d * d, axis=1))
    hard = hard_ref[0, :]
    lg = lg_ref[0, :]
    w = inl_ref[0, :]
    pos = jnp.maximum(1.0 + dp - hard, 0.0)
    neg = jnp.maximum(1.0 - dp, 0.0)
    sums = [
        jnp.sum(pos * lg * w), jnp.sum(pos * lg),
        jnp.sum(neg * lg * w), jnp.sum(neg * lg),
        jnp.sum(lg * w), jnp.sum(lg), jnp.sum(w),
    ]
    for n, s in enumerate(sums):
        prev = jnp.where(k == 0, 0.0, acc[n])
        acc[n] = prev + s
    @pl.when(k == _GC - 1)
    def _():
        use_ones = acc[6] < 8.0
        pos_s = jnp.where(use_ones, acc[1], acc[0])
        neg_s = jnp.where(use_ones, acc[3], acc[2])
        lg_s = jnp.where(use_ones, acc[5], acc[4])
        num = jnp.where(lab_ref[0] != 0, pos_s, neg_s)
        out_ref[0] = num / (lg_s + 1e-8)


def _loss_call(a_rows, b_rows, hard, lg, inl, label, interpret=False):
    return pl.pallas_call(
        _loss_body,
        grid=(_GC,),
        in_specs=[
            pl.BlockSpec((_RB, D), lambda k: (k, 0)),
            pl.BlockSpec((_RB, D), lambda k: (k, 0)),
            pl.BlockSpec((1, _RB), lambda k: (0, k)),
            pl.BlockSpec((1, _RB), lambda k: (0, k)),
            pl.BlockSpec((1, _RB), lambda k: (0, k)),
            pl.BlockSpec(memory_space=pltpu.SMEM),
        ],
        out_specs=pl.BlockSpec(memory_space=pltpu.SMEM),
        out_shape=jax.ShapeDtypeStruct((1,), jnp.float32),
        scratch_shapes=[pltpu.SMEM((7,), jnp.float32)],
        compiler_params=pltpu.CompilerParams(
            dimension_semantics=("arbitrary",)),
        interpret=interpret,
    )(a_rows, b_rows, hard, lg, inl, label)


def kernel(desc1, desc2, matches, inliers, label, logits_1, logits_2):
    mflat = matches.reshape(-1)
    snn1, snn2 = _snn_call(desc1, desc2)
    a_rows, b_rows = _rowgather_call(mflat, desc1, desc2)
    hard, lgm = _elemgather_call(
        mflat, snn1.reshape(-1), snn2.reshape(-1), logits_1, logits_2)
    out = _loss_call(
        a_rows, b_rows, hard.reshape(1, M), lgm.reshape(1, M),
        inliers.astype(jnp.float32).reshape(1, M),
        jnp.asarray(label, jnp.int32).reshape(1))
    return out.reshape(())
